# baseline (speedup 1.0000x reference)
import jax
import jax.numpy as jnp
from jax.experimental import pallas as pl
from jax.experimental.pallas import tpu as pltpu

N = 10000
D_IN = 128
DIM = 32
BLK1 = 400
BLK2 = 2000


def _pass1_kernel(A_ref, w_aw_ref, x_ref, w_axw_ref, W_att_ref,
                  y_ref, A8_ref, wcat_ref):
    @pl.when(pl.program_id(0) == 0)
    def _():
        wcat_ref[:, :DIM] = w_aw_ref[:].astype(jnp.bfloat16)
        wcat_ref[:, DIM:] = jnp.dot(
            x_ref[:], w_axw_ref[:],
            preferred_element_type=jnp.float32).astype(jnp.bfloat16)

    A_blk = A_ref[:]
    A8_ref[:] = A_blk.astype(jnp.float8_e4m3fn)
    h = jnp.dot(A_blk.astype(jnp.bfloat16), wcat_ref[:],
                preferred_element_type=jnp.float32)
    a = jnp.maximum(h[:, :DIM], 0.0)
    b = jnp.maximum(h[:, DIM:], 0.0)
    c = (a + b) * 0.5
    context = jnp.tanh(
        jnp.dot(c, W_att_ref[:], preferred_element_type=jnp.float32))
    s1 = jax.nn.sigmoid(jnp.sum(a * context, axis=1, keepdims=True)) + 1e-10
    s2 = jax.nn.sigmoid(jnp.sum(b * context, axis=1, keepdims=True)) + 1e-10
    inv = jax.lax.rsqrt(s1 * s1 + s2 * s2)
    y = a * (s1 * inv) + b * (s2 * inv)
    y_ref[:] = y.astype(jnp.bfloat16)


def _pass2_kernel(A8_ref, y_ref, out_ref):
    out_ref[:] = jnp.dot(A8_ref[:].astype(jnp.bfloat16), y_ref[:],
                         preferred_element_type=jnp.float32)


def kernel(x, A, w_aw, w_axw, W_att):
    y, A8 = pl.pallas_call(
        _pass1_kernel,
        grid=(N // BLK1,),
        in_specs=[
            pl.BlockSpec((BLK1, N), lambda i: (i, 0)),
            pl.BlockSpec((N, DIM), lambda i: (0, 0)),
            pl.BlockSpec((N, D_IN), lambda i: (0, 0)),
            pl.BlockSpec((D_IN, DIM), lambda i: (0, 0)),
            pl.BlockSpec((DIM, DIM), lambda i: (0, 0)),
        ],
        out_specs=[
            pl.BlockSpec((BLK1, DIM), lambda i: (i, 0)),
            pl.BlockSpec((BLK1, N), lambda i: (i, 0)),
        ],
        out_shape=[
            jax.ShapeDtypeStruct((N, DIM), jnp.bfloat16),
            jax.ShapeDtypeStruct((N, N), jnp.float8_e4m3fn),
        ],
        scratch_shapes=[pltpu.VMEM((N, 2 * DIM), jnp.bfloat16)],
        compiler_params=pltpu.CompilerParams(
            dimension_semantics=("arbitrary",),
            vmem_limit_bytes=128 * 1024 * 1024),
    )(A, w_aw, x, w_axw, W_att)

    out = pl.pallas_call(
        _pass2_kernel,
        grid=(N // BLK2,),
        in_specs=[
            pl.BlockSpec((BLK2, N), lambda i: (i, 0)),
            pl.BlockSpec((N, DIM), lambda i: (0, 0)),
        ],
        out_specs=pl.BlockSpec((BLK2, DIM), lambda i: (i, 0)),
        out_shape=jax.ShapeDtypeStruct((N, DIM), jnp.float32),
        compiler_params=pltpu.CompilerParams(
            dimension_semantics=("arbitrary",),
            vmem_limit_bytes=128 * 1024 * 1024),
    )(A8, y)

    return out


# speedup vs baseline: 1.0138x; 1.0138x over previous
import jax
import jax.numpy as jnp
from jax.experimental import pallas as pl
from jax.experimental.pallas import tpu as pltpu

N = 10000
D_IN = 128
DIM = 32
BLK1 = 512
NBLK1 = (N + BLK1 - 1) // BLK1
BLK2 = 2000


def _pass1_kernel(A_ref, w_awT_ref, x_ref, w_axw_ref, W_att_ref,
                  yT_ref, A8_ref, wcatT_ref):
    i = pl.program_id(0)

    @pl.when(i == 0)
    def _():
        wcatT_ref[:DIM, :] = w_awT_ref[:].astype(jnp.bfloat16)
        xw = jnp.dot(x_ref[:], w_axw_ref[:],
                     preferred_element_type=jnp.float32)
        wcatT_ref[DIM:, :] = xw.T.astype(jnp.bfloat16)

    A_blk = A_ref[:]
    A8_ref[:] = A_blk.astype(jnp.float8_e4m3fn)
    h = jax.lax.dot_general(
        A_blk.astype(jnp.bfloat16), wcatT_ref[:],
        (((1,), (1,)), ((), ())), preferred_element_type=jnp.float32)
    a = jnp.maximum(h[:, :DIM], 0.0)
    b = jnp.maximum(h[:, DIM:], 0.0)
    c = (a + b) * 0.5
    context = jnp.tanh(
        jnp.dot(c, W_att_ref[:], preferred_element_type=jnp.float32))
    s1 = jax.nn.sigmoid(jnp.sum(a * context, axis=1, keepdims=True)) + 1e-10
    s2 = jax.nn.sigmoid(jnp.sum(b * context, axis=1, keepdims=True)) + 1e-10
    inv = jax.lax.rsqrt(s1 * s1 + s2 * s2)
    y = a * (s1 * inv) + b * (s2 * inv)
    yT_ref[:, pl.ds(i * BLK1, BLK1)] = y.T.astype(jnp.bfloat16)


def _pass2_kernel(A8_ref, yT_ref, out_ref):
    out_ref[:] = jax.lax.dot_general(
        A8_ref[:].astype(jnp.bfloat16), yT_ref[:, pl.ds(0, N)],
        (((1,), (1,)), ((), ())), preferred_element_type=jnp.float32)


def kernel(x, A, w_aw, w_axw, W_att):
    yT, A8 = pl.pallas_call(
        _pass1_kernel,
        grid=(NBLK1,),
        in_specs=[
            pl.BlockSpec((BLK1, N), lambda i: (i, 0)),
            pl.BlockSpec((DIM, N), lambda i: (0, 0)),
            pl.BlockSpec((N, D_IN), lambda i: (0, 0)),
            pl.BlockSpec((D_IN, DIM), lambda i: (0, 0)),
            pl.BlockSpec((DIM, DIM), lambda i: (0, 0)),
        ],
        out_specs=[
            pl.BlockSpec((DIM, NBLK1 * BLK1), lambda i: (0, 0)),
            pl.BlockSpec((BLK1, N), lambda i: (i, 0)),
        ],
        out_shape=[
            jax.ShapeDtypeStruct((DIM, NBLK1 * BLK1), jnp.bfloat16),
            jax.ShapeDtypeStruct((N, N), jnp.float8_e4m3fn),
        ],
        scratch_shapes=[pltpu.VMEM((2 * DIM, N), jnp.bfloat16)],
        compiler_params=pltpu.CompilerParams(
            dimension_semantics=("arbitrary",),
            vmem_limit_bytes=128 * 1024 * 1024),
    )(A, w_aw.T, x, w_axw, W_att)

    out = pl.pallas_call(
        _pass2_kernel,
        grid=(N // BLK2,),
        in_specs=[
            pl.BlockSpec((BLK2, N), lambda i: (i, 0)),
            pl.BlockSpec((DIM, NBLK1 * BLK1), lambda i: (0, 0)),
        ],
        out_specs=pl.BlockSpec((BLK2, DIM), lambda i: (i, 0)),
        out_shape=jax.ShapeDtypeStruct((N, DIM), jnp.float32),
        compiler_params=pltpu.CompilerParams(
            dimension_semantics=("parallel",),
            vmem_limit_bytes=128 * 1024 * 1024),
    )(A8, yT)

    return out
